# baseline (speedup 1.0000x reference)
import functools

import jax
import jax.numpy as jnp
from jax import lax
from jax.experimental import pallas as pl
from jax.experimental.pallas import tpu as pltpu
from jax.experimental.pallas import tpu_sc as plsc

N_NODES = 10000
N_EDGES = 160000
D_FEAT = 256
D_EMB_HID = 256
D_EMB_OUT = 128
D_ENC = 64
D_EDGE = 16
D_CLS_HID = 128
N_CLASSES = 8

NC = 2
NS = 16
NW = NC * NS
EPW = N_EDGES // NW
SUP = 512
GRP = 128
NGA = SUP // GRP
CHUNKS = [(k * SUP, SUP) for k in range(EPW // SUP)]
CHUNKS.append(((EPW // SUP) * SUP, EPW - (EPW // SUP) * SUP))
GTL = CHUNKS[-1][1] - (CHUNKS[-1][1] // GRP) * GRP
STR = 624
REM0 = NS * STR
REMN = N_NODES - REM0
ZR = 128


def _elu(v):
    return jnp.where(v > 0, v, jnp.exp(jnp.minimum(v, 0.0)) - 1.0)


def _tc1_body(x_ref, w1_ref, b1_ref, w2_ref, b2_ref, wn_ref, ws_ref,
              p_ref, sf_ref):
    h1 = _elu(jnp.dot(x_ref[...], w1_ref[...],
                      preferred_element_type=jnp.float32) + b1_ref[...])
    h = jnp.dot(h1, w2_ref[...], preferred_element_type=jnp.float32) + b2_ref[...]
    p_ref[...] = jnp.dot(h, wn_ref[...], preferred_element_type=jnp.float32)
    sf_ref[...] = jnp.dot(h, ws_ref[...], preferred_element_type=jnp.float32)


_M_TILE1 = 2000


def _tc1(x, w1, b1, w2, b2, wn, ws):
    grid = (N_NODES // _M_TILE1,)
    full = lambda i: (0, 0)
    return pl.pallas_call(
        _tc1_body,
        grid=grid,
        in_specs=[
            pl.BlockSpec((_M_TILE1, D_FEAT), lambda i: (i, 0)),
            pl.BlockSpec((D_FEAT, D_EMB_HID), full),
            pl.BlockSpec((1, D_EMB_HID), full),
            pl.BlockSpec((D_EMB_HID, D_EMB_OUT), full),
            pl.BlockSpec((1, D_EMB_OUT), full),
            pl.BlockSpec((D_EMB_OUT, D_ENC), full),
            pl.BlockSpec((D_EMB_OUT, D_ENC), full),
        ],
        out_specs=[
            pl.BlockSpec((_M_TILE1, D_ENC), lambda i: (i, 0)),
            pl.BlockSpec((_M_TILE1, D_ENC), lambda i: (i, 0)),
        ],
        out_shape=[
            jax.ShapeDtypeStruct((N_NODES, D_ENC), jnp.float32),
            jax.ShapeDtypeStruct((N_NODES, D_ENC), jnp.float32),
        ],
    )(x, w1, b1, w2, b2, wn, ws)


def _sc_body(p_hbm, src_hbm, dst_hbm, ea_hbm, agg_out, ea_out,
             src_v, dst_b, dstt_b, rows_v, ea_b, ea_c, agg_s, ea_s,
             sem_i, sem_g, sem_s):
    c = lax.axis_index("c")
    s = lax.axis_index("s")
    wid = s * NC + c
    base = wid * EPW

    def zrow(r, carry):
        for q in range(D_ENC // 16):
            rows_v[r, pl.ds(q * 16, 16)] = jnp.zeros((16,), jnp.float32)
        ea_c[r, pl.ds(0, 16)] = jnp.zeros((16,), jnp.float32)
        return carry

    lax.fori_loop(0, ZR, zrow, 0)
    zrows = rows_v.at[pl.ds(0, ZR)]
    zea = ea_c.at[pl.ds(0, ZR)]
    row0 = s * STR
    z_d = [pltpu.async_copy(src_hbm.at[pl.ds(base, EPW)], src_v, sem_g)]
    for t in range(STR // ZR):
        z_d.append(pltpu.async_copy(
            zrows, agg_s.at[pl.ds(row0 + t * ZR, ZR)], sem_i))
        z_d.append(pltpu.async_copy(
            zea, ea_s.at[pl.ds(row0 + t * ZR, ZR)], sem_i))
    rem = STR % ZR
    rbase = row0 + (STR // ZR) * ZR
    z_d.append(pltpu.async_copy(
        rows_v.at[pl.ds(0, rem)], agg_s.at[pl.ds(rbase, rem)], sem_i))
    z_d.append(pltpu.async_copy(
        ea_c.at[pl.ds(0, rem)], ea_s.at[pl.ds(rbase, rem)], sem_i))

    @pl.when(s == NS - 1)
    def _():
        pltpu.sync_copy(rows_v.at[pl.ds(0, REMN)], agg_s.at[pl.ds(REM0, REMN)])
        pltpu.sync_copy(ea_c.at[pl.ds(0, REMN)],
                        ea_s.at[pl.ds(REM0, REMN)])

    for d in z_d:
        d.wait()
    plsc.subcore_barrier()

    def issue_inputs(off, sz, b):
        d = []
        for t in range(sz // GRP):
            d.append(pltpu.async_copy(
                dst_hbm.at[pl.ds(base + off + t * GRP, GRP)],
                dst_b.at[b, t], sem_i))
        if sz % GRP:
            d.append(pltpu.async_copy(
                dst_hbm.at[pl.ds(base + off + (sz // GRP) * GRP, sz % GRP)],
                dstt_b.at[b], sem_i))
        d.append(pltpu.async_copy(
            ea_hbm.at[:, pl.ds(base + off, sz)],
            ea_b.at[b, :, pl.ds(0, sz)], sem_i))
        return d

    iota16 = lax.broadcasted_iota(jnp.int32, (16,), 0)

    def transpose_ea(b, sz):
        n16 = sz // 16

        def tpose(g, carry):
            rows = g * 16 + iota16
            for f in range(D_EDGE):
                vals = ea_b[b, f, pl.ds(g * 16, 16)]
                plsc.store_scatter(
                    ea_c, [rows, jnp.full((16,), f, jnp.int32)], vals)
            return carry

        lax.fori_loop(0, n16, tpose, 0)
        if sz % 16:
            rows = n16 * 16 + iota16
            msk = iota16 < (sz % 16)
            for f in range(D_EDGE):
                vals = ea_b[b, f, pl.ds(n16 * 16, 16)]
                plsc.store_scatter(
                    ea_c, [rows, jnp.full((16,), f, jnp.int32)], vals,
                    mask=msk)

    in_d = issue_inputs(*CHUNKS[0], 0)
    sc_d = []
    for k, (off, sz) in enumerate(CHUNKS):
        b = k % 2
        for d in sc_d:
            d.wait()
        sc_d = []
        if k + 1 < len(CHUNKS):
            nxt = issue_inputs(*CHUNKS[k + 1], 1 - b)
        else:
            nxt = []
        g_d = pltpu.async_copy(p_hbm.at[src_v.at[pl.ds(off, sz)]],
                               rows_v.at[pl.ds(0, sz)], sem_g)
        for d in in_d:
            d.wait()
        in_d = nxt
        transpose_ea(b, sz)
        for t in range(sz // GRP):
            sc_d.append(pltpu.async_copy(
                ea_c.at[pl.ds(t * GRP, GRP)],
                ea_s.at[dst_b.at[b, t]], sem_s, add=True))
        if sz % GRP:
            g0 = (sz // GRP) * GRP
            sc_d.append(pltpu.async_copy(
                ea_c.at[pl.ds(g0, sz % GRP)],
                ea_s.at[dstt_b.at[b]], sem_s, add=True))
        g_d.wait()
        for t in range(sz // GRP):
            sc_d.append(pltpu.async_copy(
                rows_v.at[pl.ds(t * GRP, GRP)],
                agg_s.at[dst_b.at[b, t]], sem_s, add=True))
        if sz % GRP:
            g0 = (sz // GRP) * GRP
            sc_d.append(pltpu.async_copy(
                rows_v.at[pl.ds(g0, sz % GRP)],
                agg_s.at[dstt_b.at[b]], sem_s, add=True))
    for d in sc_d:
        d.wait()

    plsc.subcore_barrier()
    o_d = [
        pltpu.async_copy(agg_s.at[pl.ds(row0, STR)],
                         agg_out.at[c, pl.ds(row0, STR)], sem_g),
        pltpu.async_copy(ea_s.at[pl.ds(row0, STR)],
                         ea_out.at[c, pl.ds(row0, STR)], sem_g),
    ]

    @pl.when(s == NS - 1)
    def _():
        pltpu.sync_copy(agg_s.at[pl.ds(REM0, REMN)],
                        agg_out.at[c, pl.ds(REM0, REMN)])
        pltpu.sync_copy(ea_s.at[pl.ds(REM0, REMN)],
                        ea_out.at[c, pl.ds(REM0, REMN)])

    for d in o_d:
        d.wait()


_sc_scatter = functools.partial(
    pl.kernel,
    mesh=plsc.VectorSubcoreMesh(core_axis_name="c", subcore_axis_name="s"),
    compiler_params=pltpu.CompilerParams(use_tc_tiling_on_sc=False,
                                         needs_layout_passes=False),
    out_type=[
        jax.ShapeDtypeStruct((NC, N_NODES, D_ENC), jnp.float32),
        jax.ShapeDtypeStruct((NC, N_NODES, D_EDGE), jnp.float32),
    ],
    scratch_types=[
        pltpu.VMEM((EPW,), jnp.int32),
        pltpu.VMEM((2, NGA, GRP), jnp.int32),
        pltpu.VMEM((2, GTL), jnp.int32),
        pltpu.VMEM((SUP, D_ENC), jnp.float32),
        pltpu.VMEM((2, D_EDGE, SUP), jnp.float32),
        pltpu.VMEM((SUP, D_EDGE), jnp.float32),
        pltpu.VMEM_SHARED((N_NODES, D_ENC), jnp.float32),
        pltpu.VMEM_SHARED((N_NODES, D_EDGE), jnp.float32),
        pltpu.SemaphoreType.DMA,
        pltpu.SemaphoreType.DMA,
        pltpu.SemaphoreType.DMA,
    ],
)(_sc_body)


def _tc2_body(sf_ref, agg_ref, ea_ref, we_ref, be_ref,
              wc1_ref, bc1_ref, wc2_ref, bc2_ref, out_ref, acc_ref):
    i = pl.program_id(0)

    @pl.when(i == 0)
    def _():
        acc_ref[...] = jnp.zeros_like(acc_ref)

    nf = _elu(sf_ref[...] + agg_ref[...]
              + jnp.dot(ea_ref[...], we_ref[...],
                        preferred_element_type=jnp.float32)
              + be_ref[...])
    acc_ref[...] += jnp.sum(nf, axis=0, keepdims=True)

    @pl.when(i == pl.num_programs(0) - 1)
    def _():
        feat = acc_ref[...] * jnp.float32(1.0 / N_NODES)
        z = _elu(jnp.dot(feat, wc1_ref[...],
                         preferred_element_type=jnp.float32) + bc1_ref[...])
        logits = jnp.dot(z, wc2_ref[...],
                         preferred_element_type=jnp.float32) + bc2_ref[...]
        m = jnp.max(logits, axis=-1, keepdims=True)
        ex = jnp.exp(logits - m)
        probs = ex / jnp.sum(ex, axis=-1, keepdims=True)
        out_ref[...] = jnp.where(jnp.isnan(probs), jnp.float32(1e-6), probs)


_M_TILE2 = 2000


def _tc2(sf, agg2, ea2, we, be, wc1, bc1, wc2, bc2):
    grid = (N_NODES // _M_TILE2,)
    full = lambda i: (0, 0)
    return pl.pallas_call(
        _tc2_body,
        grid=grid,
        in_specs=[
            pl.BlockSpec((_M_TILE2, D_ENC), lambda i: (i, 0)),
            pl.BlockSpec((_M_TILE2, D_ENC), lambda i: (i, 0)),
            pl.BlockSpec((_M_TILE2, D_EDGE), lambda i: (i, 0)),
            pl.BlockSpec((D_EDGE, D_ENC), full),
            pl.BlockSpec((1, D_ENC), full),
            pl.BlockSpec((D_ENC, D_CLS_HID), full),
            pl.BlockSpec((1, D_CLS_HID), full),
            pl.BlockSpec((D_CLS_HID, N_CLASSES), full),
            pl.BlockSpec((1, N_CLASSES), full),
        ],
        out_specs=pl.BlockSpec((1, N_CLASSES), full),
        out_shape=jax.ShapeDtypeStruct((1, N_CLASSES), jnp.float32),
        scratch_shapes=[pltpu.VMEM((1, D_ENC), jnp.float32)],
    )(sf, agg2, ea2, we, be, wc1, bc1, wc2, bc2)


@jax.jit
def kernel(x, edge_index, edge_attr, batch,
           W_emb1, b_emb1, W_emb2, b_emb2,
           W_self, W_nbr, W_edge, b_enc,
           W_c1, b_c1, W_c2, b_c2):
    del batch
    src = edge_index[0].astype(jnp.int32)
    dst = edge_index[1].astype(jnp.int32)

    p, sf = _tc1(x, W_emb1, b_emb1.reshape(1, -1), W_emb2,
                 b_emb2.reshape(1, -1), W_nbr, W_self)
    agg2, ea2 = _sc_scatter(p, src, dst, edge_attr.T)
    return _tc2(sf, agg2[0] + agg2[1], ea2[0] + ea2[1],
                W_edge, b_enc.reshape(1, -1),
                W_c1, b_c1.reshape(1, -1), W_c2, b_c2.reshape(1, -1))


# speedup vs baseline: 1.0778x; 1.0778x over previous
import functools

import jax
import jax.numpy as jnp
from jax import lax
from jax.experimental import pallas as pl
from jax.experimental.pallas import tpu as pltpu
from jax.experimental.pallas import tpu_sc as plsc

N_NODES = 10000
N_EDGES = 160000
D_FEAT = 256
D_EMB_HID = 256
D_EMB_OUT = 128
D_ENC = 64
D_EDGE = 16
D_CLS_HID = 128
N_CLASSES = 8

NC = 2
NS = 16
NW = NC * NS
EPW = N_EDGES // NW
SUP = 512
GRP = 128
NGA = SUP // GRP
CHUNKS = [(k * SUP, SUP) for k in range(EPW // SUP)]
CHUNKS.append(((EPW // SUP) * SUP, EPW - (EPW // SUP) * SUP))
GTL = CHUNKS[-1][1] - (CHUNKS[-1][1] // GRP) * GRP
STR = 624
REM0 = NS * STR
REMN = N_NODES - REM0
ZR = 128


def _elu(v):
    return jnp.where(v > 0, v, jnp.exp(jnp.minimum(v, 0.0)) - 1.0)


def _tc1_body(x_ref, w1_ref, b1_ref, w2_ref, b2_ref, wn_ref, ws_ref,
              p_ref, sf_ref):
    h1 = _elu(jnp.dot(x_ref[...], w1_ref[...],
                      preferred_element_type=jnp.float32) + b1_ref[...])
    h = jnp.dot(h1, w2_ref[...], preferred_element_type=jnp.float32) + b2_ref[...]
    p_ref[...] = jnp.dot(h, wn_ref[...], preferred_element_type=jnp.float32)
    sf_ref[...] = jnp.dot(h, ws_ref[...], preferred_element_type=jnp.float32)


_M_TILE1 = 2000


def _tc1(x, w1, b1, w2, b2, wn, ws):
    grid = (N_NODES // _M_TILE1,)
    full = lambda i: (0, 0)
    return pl.pallas_call(
        _tc1_body,
        grid=grid,
        in_specs=[
            pl.BlockSpec((_M_TILE1, D_FEAT), lambda i: (i, 0)),
            pl.BlockSpec((D_FEAT, D_EMB_HID), full),
            pl.BlockSpec((1, D_EMB_HID), full),
            pl.BlockSpec((D_EMB_HID, D_EMB_OUT), full),
            pl.BlockSpec((1, D_EMB_OUT), full),
            pl.BlockSpec((D_EMB_OUT, D_ENC), full),
            pl.BlockSpec((D_EMB_OUT, D_ENC), full),
        ],
        out_specs=[
            pl.BlockSpec((_M_TILE1, D_ENC), lambda i: (i, 0)),
            pl.BlockSpec((_M_TILE1, D_ENC), lambda i: (i, 0)),
        ],
        out_shape=[
            jax.ShapeDtypeStruct((N_NODES, D_ENC), jnp.float32),
            jax.ShapeDtypeStruct((N_NODES, D_ENC), jnp.float32),
        ],
    )(x, w1, b1, w2, b2, wn, ws)


def _sc_body(p_hbm, src_hbm, dst_hbm, ea_hbm, agg_out, ea_out,
             src_v, dst_b, dstt_b, rows_v, ea_b, ea_c, agg_s, ea_s,
             sem_i, sem_g, sem_s):
    c = lax.axis_index("c")
    s = lax.axis_index("s")
    wid = s * NC + c
    base = wid * EPW

    def zrow(r, carry):
        for q in range(D_ENC // 16):
            rows_v[r, pl.ds(q * 16, 16)] = jnp.zeros((16,), jnp.float32)
        ea_c[r, pl.ds(0, 16)] = jnp.zeros((16,), jnp.float32)
        return carry

    lax.fori_loop(0, ZR, zrow, 0)
    zrows = rows_v.at[pl.ds(0, ZR)]
    zea = ea_c.at[pl.ds(0, ZR)]
    row0 = s * STR
    z_d = [pltpu.async_copy(src_hbm.at[pl.ds(base, EPW)], src_v, sem_g)]
    for t in range(STR // ZR):
        z_d.append(pltpu.async_copy(
            zrows, agg_s.at[pl.ds(row0 + t * ZR, ZR)], sem_i))
        z_d.append(pltpu.async_copy(
            zea, ea_s.at[pl.ds(row0 + t * ZR, ZR)], sem_i))
    rem = STR % ZR
    rbase = row0 + (STR // ZR) * ZR
    z_d.append(pltpu.async_copy(
        rows_v.at[pl.ds(0, rem)], agg_s.at[pl.ds(rbase, rem)], sem_i))
    z_d.append(pltpu.async_copy(
        ea_c.at[pl.ds(0, rem)], ea_s.at[pl.ds(rbase, rem)], sem_i))

    @pl.when(s == NS - 1)
    def _():
        pltpu.sync_copy(rows_v.at[pl.ds(0, REMN)], agg_s.at[pl.ds(REM0, REMN)])
        pltpu.sync_copy(ea_c.at[pl.ds(0, REMN)],
                        ea_s.at[pl.ds(REM0, REMN)])

    for d in z_d:
        d.wait()
    plsc.subcore_barrier()

    def issue_inputs(off, sz, b):
        d = []
        for t in range(sz // GRP):
            d.append(pltpu.async_copy(
                dst_hbm.at[pl.ds(base + off + t * GRP, GRP)],
                dst_b.at[b, t], sem_i))
        if sz % GRP:
            d.append(pltpu.async_copy(
                dst_hbm.at[pl.ds(base + off + (sz // GRP) * GRP, sz % GRP)],
                dstt_b.at[b], sem_i))
        d.append(pltpu.async_copy(
            ea_hbm.at[:, pl.ds(base + off, sz)],
            ea_b.at[b, :, pl.ds(0, sz)], sem_i))
        return d

    iota16 = lax.broadcasted_iota(jnp.int32, (16,), 0)

    def transpose_ea(b, sz):
        n16 = sz // 16

        def tpose(g, carry):
            rows = g * 16 + iota16
            for f in range(D_EDGE):
                vals = ea_b[b, f, pl.ds(g * 16, 16)]
                plsc.store_scatter(
                    ea_c, [rows, jnp.full((16,), f, jnp.int32)], vals)
            return carry

        lax.fori_loop(0, n16, tpose, 0)
        if sz % 16:
            rows = n16 * 16 + iota16
            msk = iota16 < (sz % 16)
            for f in range(D_EDGE):
                vals = ea_b[b, f, pl.ds(n16 * 16, 16)]
                plsc.store_scatter(
                    ea_c, [rows, jnp.full((16,), f, jnp.int32)], vals,
                    mask=msk)

    in_d = issue_inputs(*CHUNKS[0], 0)
    sc_d = []
    for k, (off, sz) in enumerate(CHUNKS):
        b = k % 2
        for d in sc_d:
            d.wait()
        sc_d = []
        if k + 1 < len(CHUNKS):
            nxt = issue_inputs(*CHUNKS[k + 1], 1 - b)
        else:
            nxt = []
        g_d = pltpu.async_copy(p_hbm.at[src_v.at[pl.ds(off, sz)]],
                               rows_v.at[pl.ds(0, sz)], sem_g)
        for d in in_d:
            d.wait()
        in_d = nxt
        transpose_ea(b, sz)
        for t in range(sz // GRP):
            sc_d.append(pltpu.async_copy(
                ea_c.at[pl.ds(t * GRP, GRP)],
                ea_s.at[dst_b.at[b, t]], sem_s, add=True))
        if sz % GRP:
            g0 = (sz // GRP) * GRP
            sc_d.append(pltpu.async_copy(
                ea_c.at[pl.ds(g0, sz % GRP)],
                ea_s.at[dstt_b.at[b]], sem_s, add=True))
        g_d.wait()
        for t in range(sz // GRP):
            sc_d.append(pltpu.async_copy(
                rows_v.at[pl.ds(t * GRP, GRP)],
                agg_s.at[dst_b.at[b, t]], sem_s, add=True))
        if sz % GRP:
            g0 = (sz // GRP) * GRP
            sc_d.append(pltpu.async_copy(
                rows_v.at[pl.ds(g0, sz % GRP)],
                agg_s.at[dstt_b.at[b]], sem_s, add=True))
    for d in sc_d:
        d.wait()

    plsc.subcore_barrier()
    o_d = [
        pltpu.async_copy(agg_s.at[pl.ds(row0, STR)],
                         agg_out.at[c, pl.ds(row0, STR)], sem_g),
        pltpu.async_copy(ea_s.at[pl.ds(row0, STR)],
                         ea_out.at[c, pl.ds(row0, STR)], sem_g),
    ]

    @pl.when(s == NS - 1)
    def _():
        pltpu.sync_copy(agg_s.at[pl.ds(REM0, REMN)],
                        agg_out.at[c, pl.ds(REM0, REMN)])
        pltpu.sync_copy(ea_s.at[pl.ds(REM0, REMN)],
                        ea_out.at[c, pl.ds(REM0, REMN)])

    for d in o_d:
        d.wait()


_sc_scatter = functools.partial(
    pl.kernel,
    mesh=plsc.VectorSubcoreMesh(core_axis_name="c", subcore_axis_name="s"),
    compiler_params=pltpu.CompilerParams(use_tc_tiling_on_sc=False,
                                         needs_layout_passes=False),
    out_type=[
        jax.ShapeDtypeStruct((NC, N_NODES, D_ENC), jnp.float32),
        jax.ShapeDtypeStruct((NC, N_NODES, D_EDGE), jnp.float32),
    ],
    scratch_types=[
        pltpu.VMEM((EPW,), jnp.int32),
        pltpu.VMEM((2, NGA, GRP), jnp.int32),
        pltpu.VMEM((2, GTL), jnp.int32),
        pltpu.VMEM((SUP, D_ENC), jnp.float32),
        pltpu.VMEM((2, D_EDGE, SUP), jnp.float32),
        pltpu.VMEM((SUP, D_EDGE), jnp.float32),
        pltpu.VMEM_SHARED((N_NODES, D_ENC), jnp.float32),
        pltpu.VMEM_SHARED((N_NODES, D_EDGE), jnp.float32),
        pltpu.SemaphoreType.DMA,
        pltpu.SemaphoreType.DMA,
        pltpu.SemaphoreType.DMA,
    ],
)(_sc_body)


def _tc2_body(sf_ref, agg_ref, ea_ref, we_ref, be_ref,
              wc1_ref, bc1_ref, wc2_ref, bc2_ref, out_ref, acc_ref):
    i = pl.program_id(0)

    @pl.when(i == 0)
    def _():
        acc_ref[...] = jnp.zeros_like(acc_ref)

    a = agg_ref[0] + agg_ref[1]
    e = ea_ref[0] + ea_ref[1]
    nf = _elu(sf_ref[...] + a
              + jnp.dot(e, we_ref[...], preferred_element_type=jnp.float32)
              + be_ref[...])
    acc_ref[...] += jnp.sum(nf, axis=0, keepdims=True)

    @pl.when(i == pl.num_programs(0) - 1)
    def _():
        feat = acc_ref[...] * jnp.float32(1.0 / N_NODES)
        z = _elu(jnp.dot(feat, wc1_ref[...],
                         preferred_element_type=jnp.float32) + bc1_ref[...])
        logits = jnp.dot(z, wc2_ref[...],
                         preferred_element_type=jnp.float32) + bc2_ref[...]
        m = jnp.max(logits, axis=-1, keepdims=True)
        ex = jnp.exp(logits - m)
        probs = ex / jnp.sum(ex, axis=-1, keepdims=True)
        out_ref[...] = jnp.where(jnp.isnan(probs), jnp.float32(1e-6), probs)


_M_TILE2 = 2000


def _tc2(sf, agg2, ea2, we, be, wc1, bc1, wc2, bc2):
    grid = (N_NODES // _M_TILE2,)
    full = lambda i: (0, 0)
    return pl.pallas_call(
        _tc2_body,
        grid=grid,
        in_specs=[
            pl.BlockSpec((_M_TILE2, D_ENC), lambda i: (i, 0)),
            pl.BlockSpec((NC, _M_TILE2, D_ENC), lambda i: (0, i, 0)),
            pl.BlockSpec((NC, _M_TILE2, D_EDGE), lambda i: (0, i, 0)),
            pl.BlockSpec((D_EDGE, D_ENC), full),
            pl.BlockSpec((1, D_ENC), full),
            pl.BlockSpec((D_ENC, D_CLS_HID), full),
            pl.BlockSpec((1, D_CLS_HID), full),
            pl.BlockSpec((D_CLS_HID, N_CLASSES), full),
            pl.BlockSpec((1, N_CLASSES), full),
        ],
        out_specs=pl.BlockSpec((1, N_CLASSES), full),
        out_shape=jax.ShapeDtypeStruct((1, N_CLASSES), jnp.float32),
        scratch_shapes=[pltpu.VMEM((1, D_ENC), jnp.float32)],
    )(sf, agg2, ea2, we, be, wc1, bc1, wc2, bc2)


@jax.jit
def kernel(x, edge_index, edge_attr, batch,
           W_emb1, b_emb1, W_emb2, b_emb2,
           W_self, W_nbr, W_edge, b_enc,
           W_c1, b_c1, W_c2, b_c2):
    del batch
    src = edge_index[0].astype(jnp.int32)
    dst = edge_index[1].astype(jnp.int32)

    p, sf = _tc1(x, W_emb1, b_emb1.reshape(1, -1), W_emb2,
                 b_emb2.reshape(1, -1), W_nbr, W_self)
    agg2, ea2 = _sc_scatter(p, src, dst, edge_attr.T)
    return _tc2(sf, agg2, ea2, W_edge, b_enc.reshape(1, -1),
                W_c1, b_c1.reshape(1, -1), W_c2, b_c2.reshape(1, -1))


# speedup vs baseline: 1.0925x; 1.0136x over previous
import functools

import jax
import jax.numpy as jnp
from jax import lax
from jax.experimental import pallas as pl
from jax.experimental.pallas import tpu as pltpu
from jax.experimental.pallas import tpu_sc as plsc

N_NODES = 10000
N_EDGES = 160000
D_FEAT = 256
D_EMB_HID = 256
D_EMB_OUT = 128
D_ENC = 64
D_EDGE = 16
D_CLS_HID = 128
N_CLASSES = 8

NC = 2
NS = 16
NW = NC * NS
EPW = N_EDGES // NW
SUP = 384
GRP = 128
NGA = SUP // GRP
CHUNKS = [(k * SUP, SUP) for k in range(EPW // SUP)]
CHUNKS.append(((EPW // SUP) * SUP, EPW - (EPW // SUP) * SUP))
GTL = CHUNKS[-1][1] - (CHUNKS[-1][1] // GRP) * GRP
STR = 624
REM0 = NS * STR
REMN = N_NODES - REM0
ZR = 128


def _elu(v):
    return jnp.where(v > 0, v, jnp.exp(jnp.minimum(v, 0.0)) - 1.0)


def _tc1_body(x_ref, w1_ref, b1_ref, w2_ref, b2_ref, wn_ref, ws_ref,
              p_ref, sf_ref):
    h1 = _elu(jnp.dot(x_ref[...], w1_ref[...],
                      preferred_element_type=jnp.float32) + b1_ref[...])
    h = jnp.dot(h1, w2_ref[...], preferred_element_type=jnp.float32) + b2_ref[...]
    p_ref[...] = jnp.dot(h, wn_ref[...], preferred_element_type=jnp.float32)
    sf_ref[...] = jnp.dot(h, ws_ref[...], preferred_element_type=jnp.float32)


_M_TILE1 = 2000


def _tc1(x, w1, b1, w2, b2, wn, ws):
    grid = (N_NODES // _M_TILE1,)
    full = lambda i: (0, 0)
    return pl.pallas_call(
        _tc1_body,
        grid=grid,
        in_specs=[
            pl.BlockSpec((_M_TILE1, D_FEAT), lambda i: (i, 0)),
            pl.BlockSpec((D_FEAT, D_EMB_HID), full),
            pl.BlockSpec((1, D_EMB_HID), full),
            pl.BlockSpec((D_EMB_HID, D_EMB_OUT), full),
            pl.BlockSpec((1, D_EMB_OUT), full),
            pl.BlockSpec((D_EMB_OUT, D_ENC), full),
            pl.BlockSpec((D_EMB_OUT, D_ENC), full),
        ],
        out_specs=[
            pl.BlockSpec((_M_TILE1, D_ENC), lambda i: (i, 0)),
            pl.BlockSpec((_M_TILE1, D_ENC), lambda i: (i, 0)),
        ],
        out_shape=[
            jax.ShapeDtypeStruct((N_NODES, D_ENC), jnp.float32),
            jax.ShapeDtypeStruct((N_NODES, D_ENC), jnp.float32),
        ],
    )(x, w1, b1, w2, b2, wn, ws)


def _sc_body(p_hbm, src_hbm, dst_hbm, ea_hbm, agg_out, ea_out,
             src_v, dst_b, dstt_b, rows_b, ea_b, ea_c, agg_s, ea_s,
             sem_i, sem_g, sem_s):
    c = lax.axis_index("c")
    s = lax.axis_index("s")
    wid = s * NC + c
    base = wid * EPW

    def zrow(r, carry):
        for q in range(D_ENC // 16):
            rows_b[0, r, pl.ds(q * 16, 16)] = jnp.zeros((16,), jnp.float32)
        ea_c[0, r, pl.ds(0, 16)] = jnp.zeros((16,), jnp.float32)
        return carry

    lax.fori_loop(0, ZR, zrow, 0)
    zrows = rows_b.at[0, pl.ds(0, ZR)]
    zea = ea_c.at[0, pl.ds(0, ZR)]
    row0 = s * STR
    z_d = [pltpu.async_copy(src_hbm.at[pl.ds(base, EPW)], src_v, sem_g)]
    for t in range(STR // ZR):
        z_d.append(pltpu.async_copy(
            zrows, agg_s.at[pl.ds(row0 + t * ZR, ZR)], sem_i))
        z_d.append(pltpu.async_copy(
            zea, ea_s.at[pl.ds(row0 + t * ZR, ZR)], sem_i))
    rem = STR % ZR
    rbase = row0 + (STR // ZR) * ZR
    z_d.append(pltpu.async_copy(
        rows_b.at[0, pl.ds(0, rem)], agg_s.at[pl.ds(rbase, rem)], sem_i))
    z_d.append(pltpu.async_copy(
        ea_c.at[0, pl.ds(0, rem)], ea_s.at[pl.ds(rbase, rem)], sem_i))

    @pl.when(s == NS - 1)
    def _():
        pltpu.sync_copy(rows_b.at[0, pl.ds(0, REMN)],
                        agg_s.at[pl.ds(REM0, REMN)])
        pltpu.sync_copy(ea_c.at[0, pl.ds(0, REMN)],
                        ea_s.at[pl.ds(REM0, REMN)])

    for d in z_d:
        d.wait()
    plsc.subcore_barrier()

    def issue_inputs(off, sz, b):
        d = []
        for t in range(sz // GRP):
            d.append(pltpu.async_copy(
                dst_hbm.at[pl.ds(base + off + t * GRP, GRP)],
                dst_b.at[b, t], sem_i))
        if sz % GRP:
            d.append(pltpu.async_copy(
                dst_hbm.at[pl.ds(base + off + (sz // GRP) * GRP, sz % GRP)],
                dstt_b.at[b], sem_i))
        d.append(pltpu.async_copy(
            ea_hbm.at[:, pl.ds(base + off, sz)],
            ea_b.at[b, :, pl.ds(0, sz)], sem_i))
        return d

    iota16 = lax.broadcasted_iota(jnp.int32, (16,), 0)

    def transpose_ea(b, sz):
        n16 = sz // 16

        def tpose(g, carry):
            rows = g * 16 + iota16
            for f in range(D_EDGE):
                vals = ea_b[b, f, pl.ds(g * 16, 16)]
                plsc.store_scatter(
                    ea_c.at[b], [rows, jnp.full((16,), f, jnp.int32)], vals)
            return carry

        lax.fori_loop(0, n16, tpose, 0)
        if sz % 16:
            rows = n16 * 16 + iota16
            msk = iota16 < (sz % 16)
            for f in range(D_EDGE):
                vals = ea_b[b, f, pl.ds(n16 * 16, 16)]
                plsc.store_scatter(
                    ea_c.at[b], [rows, jnp.full((16,), f, jnp.int32)], vals,
                    mask=msk)

    sc_prev, sc_cur = [], []
    for k, (off, sz) in enumerate(CHUNKS):
        b = k % 2
        for d in sc_prev:
            d.wait()
        sc_prev = sc_cur
        sc_cur = []
        in_d = issue_inputs(off, sz, b)
        g_d = pltpu.async_copy(p_hbm.at[src_v.at[pl.ds(off, sz)]],
                               rows_b.at[b, pl.ds(0, sz)], sem_g)
        for d in in_d:
            d.wait()
        transpose_ea(b, sz)
        for t in range(sz // GRP):
            sc_cur.append(pltpu.async_copy(
                ea_c.at[b, pl.ds(t * GRP, GRP)],
                ea_s.at[dst_b.at[b, t]], sem_s, add=True))
        if sz % GRP:
            g0 = (sz // GRP) * GRP
            sc_cur.append(pltpu.async_copy(
                ea_c.at[b, pl.ds(g0, sz % GRP)],
                ea_s.at[dstt_b.at[b]], sem_s, add=True))
        g_d.wait()
        for t in range(sz // GRP):
            sc_cur.append(pltpu.async_copy(
                rows_b.at[b, pl.ds(t * GRP, GRP)],
                agg_s.at[dst_b.at[b, t]], sem_s, add=True))
        if sz % GRP:
            g0 = (sz // GRP) * GRP
            sc_cur.append(pltpu.async_copy(
                rows_b.at[b, pl.ds(g0, sz % GRP)],
                agg_s.at[dstt_b.at[b]], sem_s, add=True))
    for d in sc_prev:
        d.wait()
    for d in sc_cur:
        d.wait()

    plsc.subcore_barrier()
    o_d = [
        pltpu.async_copy(agg_s.at[pl.ds(row0, STR)],
                         agg_out.at[c, pl.ds(row0, STR)], sem_g),
        pltpu.async_copy(ea_s.at[pl.ds(row0, STR)],
                         ea_out.at[c, pl.ds(row0, STR)], sem_g),
    ]

    @pl.when(s == NS - 1)
    def _():
        pltpu.sync_copy(agg_s.at[pl.ds(REM0, REMN)],
                        agg_out.at[c, pl.ds(REM0, REMN)])
        pltpu.sync_copy(ea_s.at[pl.ds(REM0, REMN)],
                        ea_out.at[c, pl.ds(REM0, REMN)])

    for d in o_d:
        d.wait()


_sc_scatter = functools.partial(
    pl.kernel,
    mesh=plsc.VectorSubcoreMesh(core_axis_name="c", subcore_axis_name="s"),
    compiler_params=pltpu.CompilerParams(use_tc_tiling_on_sc=False,
                                         needs_layout_passes=False),
    out_type=[
        jax.ShapeDtypeStruct((NC, N_NODES, D_ENC), jnp.float32),
        jax.ShapeDtypeStruct((NC, N_NODES, D_EDGE), jnp.float32),
    ],
    scratch_types=[
        pltpu.VMEM((EPW,), jnp.int32),
        pltpu.VMEM((2, NGA, GRP), jnp.int32),
        pltpu.VMEM((2, GTL), jnp.int32),
        pltpu.VMEM((2, SUP, D_ENC), jnp.float32),
        pltpu.VMEM((2, D_EDGE, SUP), jnp.float32),
        pltpu.VMEM((2, SUP, D_EDGE), jnp.float32),
        pltpu.VMEM_SHARED((N_NODES, D_ENC), jnp.float32),
        pltpu.VMEM_SHARED((N_NODES, D_EDGE), jnp.float32),
        pltpu.SemaphoreType.DMA,
        pltpu.SemaphoreType.DMA,
        pltpu.SemaphoreType.DMA,
    ],
)(_sc_body)


def _tc2_body(sf_ref, agg_ref, ea_ref, we_ref, be_ref,
              wc1_ref, bc1_ref, wc2_ref, bc2_ref, out_ref, acc_ref):
    i = pl.program_id(0)

    @pl.when(i == 0)
    def _():
        acc_ref[...] = jnp.zeros_like(acc_ref)

    a = agg_ref[0] + agg_ref[1]
    e = ea_ref[0] + ea_ref[1]
    nf = _elu(sf_ref[...] + a
              + jnp.dot(e, we_ref[...], preferred_element_type=jnp.float32)
              + be_ref[...])
    acc_ref[...] += jnp.sum(nf, axis=0, keepdims=True)

    @pl.when(i == pl.num_programs(0) - 1)
    def _():
        feat = acc_ref[...] * jnp.float32(1.0 / N_NODES)
        z = _elu(jnp.dot(feat, wc1_ref[...],
                         preferred_element_type=jnp.float32) + bc1_ref[...])
        logits = jnp.dot(z, wc2_ref[...],
                         preferred_element_type=jnp.float32) + bc2_ref[...]
        m = jnp.max(logits, axis=-1, keepdims=True)
        ex = jnp.exp(logits - m)
        probs = ex / jnp.sum(ex, axis=-1, keepdims=True)
        out_ref[...] = jnp.where(jnp.isnan(probs), jnp.float32(1e-6), probs)


_M_TILE2 = 2000


def _tc2(sf, agg2, ea2, we, be, wc1, bc1, wc2, bc2):
    grid = (N_NODES // _M_TILE2,)
    full = lambda i: (0, 0)
    return pl.pallas_call(
        _tc2_body,
        grid=grid,
        in_specs=[
            pl.BlockSpec((_M_TILE2, D_ENC), lambda i: (i, 0)),
            pl.BlockSpec((NC, _M_TILE2, D_ENC), lambda i: (0, i, 0)),
            pl.BlockSpec((NC, _M_TILE2, D_EDGE), lambda i: (0, i, 0)),
            pl.BlockSpec((D_EDGE, D_ENC), full),
            pl.BlockSpec((1, D_ENC), full),
            pl.BlockSpec((D_ENC, D_CLS_HID), full),
            pl.BlockSpec((1, D_CLS_HID), full),
            pl.BlockSpec((D_CLS_HID, N_CLASSES), full),
            pl.BlockSpec((1, N_CLASSES), full),
        ],
        out_specs=pl.BlockSpec((1, N_CLASSES), full),
        out_shape=jax.ShapeDtypeStruct((1, N_CLASSES), jnp.float32),
        scratch_shapes=[pltpu.VMEM((1, D_ENC), jnp.float32)],
    )(sf, agg2, ea2, we, be, wc1, bc1, wc2, bc2)


@jax.jit
def kernel(x, edge_index, edge_attr, batch,
           W_emb1, b_emb1, W_emb2, b_emb2,
           W_self, W_nbr, W_edge, b_enc,
           W_c1, b_c1, W_c2, b_c2):
    del batch
    src = edge_index[0].astype(jnp.int32)
    dst = edge_index[1].astype(jnp.int32)

    p, sf = _tc1(x, W_emb1, b_emb1.reshape(1, -1), W_emb2,
                 b_emb2.reshape(1, -1), W_nbr, W_self)
    agg2, ea2 = _sc_scatter(p, src, dst, edge_attr.T)
    return _tc2(sf, agg2, ea2, W_edge, b_enc.reshape(1, -1),
                W_c1, b_c1.reshape(1, -1), W_c2, b_c2.reshape(1, -1))
